# baseline (speedup 1.0000x reference)
import jax
import jax.numpy as jnp
from jax.experimental import pallas as pl
from jax.experimental.pallas import tpu as pltpu

_TINY = 1e-30
_LN2 = 0.6931471805599453
_INV_LN2 = 1.4426950408889634
_ROWS = 8192
_COLS = 4096
_CHUNK = 128
_NCHUNK = _ROWS // _CHUNK
_DEPTH = 8
_CR = 8
_SCALE = 0.5 * _INV_LN2 / _ROWS


def _start(p_hbm, w_hbm, pbuf, wbuf, psem, wsem, c, slot):
    pltpu.make_async_copy(
        p_hbm.at[pl.ds(c * _CHUNK, _CHUNK), :], pbuf.at[slot], psem.at[slot]
    ).start()
    pltpu.make_async_copy(
        w_hbm.at[pl.ds(c * _CHUNK, _CHUNK), :], wbuf.at[slot], wsem.at[slot]
    ).start()


def _jsd_ring_kernel(p_hbm, w_hbm, out_ref, pbuf, wbuf, psem, wsem):
    q = _COLS // 4

    for k in range(_DEPTH):
        _start(p_hbm, w_hbm, pbuf, wbuf, psem, wsem, k, k)

    def body(c, acc):
        slot = jax.lax.rem(c, _DEPTH)
        pltpu.make_async_copy(
            p_hbm.at[pl.ds(0, _CHUNK), :], pbuf.at[slot], psem.at[slot]
        ).wait()
        pltpu.make_async_copy(
            w_hbm.at[pl.ds(0, _CHUNK), :], wbuf.at[slot], wsem.at[slot]
        ).wait()
        for r in range(0, _CHUNK, _CR):
            for cc in range(0, _COLS, q):
                p = pbuf[slot, r:r + _CR, cc:cc + q]
                w = wbuf[slot, r:r + _CR, cc:cc + q]
                s = w + p
                t = w * jnp.log(jnp.maximum(w, _TINY))
                t = t + p * jnp.log(jnp.maximum(p, _TINY))
                t = t + s * (_LN2 - jnp.log(jnp.maximum(s, _TINY)))
                acc = acc + t

        @pl.when(c + _DEPTH < _NCHUNK)
        def _prefetch():
            _start(p_hbm, w_hbm, pbuf, wbuf, psem, wsem, c + _DEPTH, slot)

        return acc

    acc = jax.lax.fori_loop(
        0, _NCHUNK, body, jnp.zeros((_CR, q), jnp.float32))
    out_ref[...] = jnp.sum(acc, keepdims=True) * _SCALE


def kernel(P, W):
    out = pl.pallas_call(
        _jsd_ring_kernel,
        grid=(1,),
        in_specs=[
            pl.BlockSpec(memory_space=pl.ANY),
            pl.BlockSpec(memory_space=pl.ANY),
        ],
        out_specs=pl.BlockSpec((1, 1), lambda i: (0, 0)),
        out_shape=jax.ShapeDtypeStruct((1, 1), jnp.float32),
        scratch_shapes=[
            pltpu.VMEM((_DEPTH, _CHUNK, _COLS), jnp.float32),
            pltpu.VMEM((_DEPTH, _CHUNK, _COLS), jnp.float32),
            pltpu.SemaphoreType.DMA((_DEPTH,)),
            pltpu.SemaphoreType.DMA((_DEPTH,)),
        ],
        compiler_params=pltpu.CompilerParams(
            dimension_semantics=("arbitrary",)
        ),
    )(P, W)
    return out.reshape(())


# speedup vs baseline: 1.0356x; 1.0356x over previous
import jax
import jax.numpy as jnp
from jax.experimental import pallas as pl
from jax.experimental.pallas import tpu as pltpu

_TINY = 1e-30
_LN2 = 0.6931471805599453
_INV_LN2 = 1.4426950408889634
_ROWS = 8192
_COLS = 4096
_CHUNK = 512
_NCHUNK = _ROWS // _CHUNK
_DEPTH = 3
_CR = 8
_SCALE = 0.5 * _INV_LN2 / _ROWS


def _start(p_hbm, w_hbm, pbuf, wbuf, psem, wsem, c, slot):
    pltpu.make_async_copy(
        p_hbm.at[pl.ds(c * _CHUNK, _CHUNK), :], pbuf.at[slot], psem.at[slot]
    ).start()
    pltpu.make_async_copy(
        w_hbm.at[pl.ds(c * _CHUNK, _CHUNK), :], wbuf.at[slot], wsem.at[slot]
    ).start()


def _jsd_ring_kernel(p_hbm, w_hbm, out_ref, pbuf, wbuf, psem, wsem):
    q = _COLS // 4

    for k in range(_DEPTH):
        _start(p_hbm, w_hbm, pbuf, wbuf, psem, wsem, k, k)

    def body(c, acc):
        slot = jax.lax.rem(c, _DEPTH)
        pltpu.make_async_copy(
            p_hbm.at[pl.ds(0, _CHUNK), :], pbuf.at[slot], psem.at[slot]
        ).wait()
        pltpu.make_async_copy(
            w_hbm.at[pl.ds(0, _CHUNK), :], wbuf.at[slot], wsem.at[slot]
        ).wait()
        for r in range(0, _CHUNK, _CR):
            for cc in range(0, _COLS, q):
                p = pbuf[slot, r:r + _CR, cc:cc + q]
                w = wbuf[slot, r:r + _CR, cc:cc + q]
                s = w + p
                t = w * jnp.log(jnp.maximum(w, _TINY))
                t = t + p * jnp.log(jnp.maximum(p, _TINY))
                t = t + s * (_LN2 - jnp.log(jnp.maximum(s, _TINY)))
                acc = acc + t

        @pl.when(c + _DEPTH < _NCHUNK)
        def _prefetch():
            _start(p_hbm, w_hbm, pbuf, wbuf, psem, wsem, c + _DEPTH, slot)

        return acc

    acc = jax.lax.fori_loop(
        0, _NCHUNK, body, jnp.zeros((_CR, q), jnp.float32))
    out_ref[...] = jnp.sum(acc, keepdims=True) * _SCALE


def kernel(P, W):
    out = pl.pallas_call(
        _jsd_ring_kernel,
        grid=(1,),
        in_specs=[
            pl.BlockSpec(memory_space=pl.ANY),
            pl.BlockSpec(memory_space=pl.ANY),
        ],
        out_specs=pl.BlockSpec((1, 1), lambda i: (0, 0)),
        out_shape=jax.ShapeDtypeStruct((1, 1), jnp.float32),
        scratch_shapes=[
            pltpu.VMEM((_DEPTH, _CHUNK, _COLS), jnp.float32),
            pltpu.VMEM((_DEPTH, _CHUNK, _COLS), jnp.float32),
            pltpu.SemaphoreType.DMA((_DEPTH,)),
            pltpu.SemaphoreType.DMA((_DEPTH,)),
        ],
        compiler_params=pltpu.CompilerParams(
            dimension_semantics=("arbitrary",)
        ),
    )(P, W)
    return out.reshape(())


# speedup vs baseline: 1.0513x; 1.0152x over previous
import jax
import jax.numpy as jnp
from jax.experimental import pallas as pl
from jax.experimental.pallas import tpu as pltpu

_TINY = 1e-30
_LN2 = 0.6931471805599453
_INV_LN2 = 1.4426950408889634
_ROWS = 8192
_COLS = 4096
_CHUNK = 256
_NCHUNK = _ROWS // _CHUNK
_DEPTH = 4
_CR = 8
_SCALE = 0.5 * _INV_LN2 / _ROWS


def _start(p_hbm, w_hbm, pbuf, wbuf, psem, wsem, c, slot):
    pltpu.make_async_copy(
        p_hbm.at[pl.ds(c * _CHUNK, _CHUNK), :], pbuf.at[slot], psem.at[slot]
    ).start()
    pltpu.make_async_copy(
        w_hbm.at[pl.ds(c * _CHUNK, _CHUNK), :], wbuf.at[slot], wsem.at[slot]
    ).start()


def _jsd_ring_kernel(p_hbm, w_hbm, out_ref, pbuf, wbuf, psem, wsem):
    q = _COLS // 4

    for k in range(_DEPTH):
        _start(p_hbm, w_hbm, pbuf, wbuf, psem, wsem, k, k)

    def body(c, acc):
        slot = jax.lax.rem(c, _DEPTH)
        pltpu.make_async_copy(
            p_hbm.at[pl.ds(0, _CHUNK), :], pbuf.at[slot], psem.at[slot]
        ).wait()
        pltpu.make_async_copy(
            w_hbm.at[pl.ds(0, _CHUNK), :], wbuf.at[slot], wsem.at[slot]
        ).wait()
        for r in range(0, _CHUNK, _CR):
            for cc in range(0, _COLS, q):
                p = pbuf[slot, r:r + _CR, cc:cc + q]
                w = wbuf[slot, r:r + _CR, cc:cc + q]
                s = w + p
                t = w * jnp.log(jnp.maximum(w, _TINY))
                t = t + p * jnp.log(jnp.maximum(p, _TINY))
                t = t + s * (_LN2 - jnp.log(jnp.maximum(s, _TINY)))
                acc = acc + t

        @pl.when(c + _DEPTH < _NCHUNK)
        def _prefetch():
            _start(p_hbm, w_hbm, pbuf, wbuf, psem, wsem, c + _DEPTH, slot)

        return acc

    acc = jax.lax.fori_loop(
        0, _NCHUNK, body, jnp.zeros((_CR, q), jnp.float32))
    out_ref[...] = jnp.sum(acc, keepdims=True) * _SCALE


def kernel(P, W):
    out = pl.pallas_call(
        _jsd_ring_kernel,
        grid=(1,),
        in_specs=[
            pl.BlockSpec(memory_space=pl.ANY),
            pl.BlockSpec(memory_space=pl.ANY),
        ],
        out_specs=pl.BlockSpec((1, 1), lambda i: (0, 0)),
        out_shape=jax.ShapeDtypeStruct((1, 1), jnp.float32),
        scratch_shapes=[
            pltpu.VMEM((_DEPTH, _CHUNK, _COLS), jnp.float32),
            pltpu.VMEM((_DEPTH, _CHUNK, _COLS), jnp.float32),
            pltpu.SemaphoreType.DMA((_DEPTH,)),
            pltpu.SemaphoreType.DMA((_DEPTH,)),
        ],
        compiler_params=pltpu.CompilerParams(
            dimension_semantics=("arbitrary",)
        ),
    )(P, W)
    return out.reshape(())


# speedup vs baseline: 1.0760x; 1.0235x over previous
import jax
import jax.numpy as jnp
from jax.experimental import pallas as pl
from jax.experimental.pallas import tpu as pltpu

_TINY = 1e-30
_LN2 = 0.6931471805599453
_INV_LN2 = 1.4426950408889634
_ROWS = 8192
_COLS = 4096
_CHUNK = 256
_NCHUNK = _ROWS // _CHUNK
_DEPTH = 5
_AHEAD = 4
_CR = 8
_SCALE = 0.5 * _INV_LN2 / _ROWS


def _start(p_hbm, w_hbm, pbuf, wbuf, psem, wsem, c, slot):
    pltpu.make_async_copy(
        p_hbm.at[pl.ds(c * _CHUNK, _CHUNK), :], pbuf.at[slot], psem.at[slot]
    ).start()
    pltpu.make_async_copy(
        w_hbm.at[pl.ds(c * _CHUNK, _CHUNK), :], wbuf.at[slot], wsem.at[slot]
    ).start()


def _jsd_ring_kernel(p_hbm, w_hbm, out_ref, pbuf, wbuf, psem, wsem):
    q = _COLS // 4

    for k in range(_AHEAD):
        _start(p_hbm, w_hbm, pbuf, wbuf, psem, wsem, k, k)

    def body(c, acc):
        slot = jax.lax.rem(c, _DEPTH)
        pltpu.make_async_copy(
            p_hbm.at[pl.ds(0, _CHUNK), :], pbuf.at[slot], psem.at[slot]
        ).wait()
        pltpu.make_async_copy(
            w_hbm.at[pl.ds(0, _CHUNK), :], wbuf.at[slot], wsem.at[slot]
        ).wait()
        @pl.when(c + _AHEAD < _NCHUNK)
        def _prefetch():
            _start(p_hbm, w_hbm, pbuf, wbuf, psem, wsem, c + _AHEAD,
                   jax.lax.rem(c + _AHEAD, _DEPTH))

        for r in range(0, _CHUNK, _CR):
            for cc in range(0, _COLS, q):
                p = pbuf[slot, r:r + _CR, cc:cc + q]
                w = wbuf[slot, r:r + _CR, cc:cc + q]
                s = w + p
                t = w * jnp.log(jnp.maximum(w, _TINY))
                t = t + p * jnp.log(jnp.maximum(p, _TINY))
                t = t + s * (_LN2 - jnp.log(jnp.maximum(s, _TINY)))
                acc = acc + t

        return acc

    acc = jax.lax.fori_loop(
        0, _NCHUNK, body, jnp.zeros((_CR, q), jnp.float32))
    out_ref[...] = jnp.sum(acc, keepdims=True) * _SCALE


def kernel(P, W):
    out = pl.pallas_call(
        _jsd_ring_kernel,
        grid=(1,),
        in_specs=[
            pl.BlockSpec(memory_space=pl.ANY),
            pl.BlockSpec(memory_space=pl.ANY),
        ],
        out_specs=pl.BlockSpec((1, 1), lambda i: (0, 0)),
        out_shape=jax.ShapeDtypeStruct((1, 1), jnp.float32),
        scratch_shapes=[
            pltpu.VMEM((_DEPTH, _CHUNK, _COLS), jnp.float32),
            pltpu.VMEM((_DEPTH, _CHUNK, _COLS), jnp.float32),
            pltpu.SemaphoreType.DMA((_DEPTH,)),
            pltpu.SemaphoreType.DMA((_DEPTH,)),
        ],
        compiler_params=pltpu.CompilerParams(
            dimension_semantics=("arbitrary",)
        ),
    )(P, W)
    return out.reshape(())
